# baseline (speedup 1.0000x reference)
import functools

import jax
import jax.numpy as jnp
from jax import lax
from jax.experimental import pallas as pl
from jax.experimental.pallas import tpu as pltpu
from jax.experimental.pallas import tpu_sc as plsc


def _sc_transpose_pack(ET):
    d, V = ET.shape
    d2 = 2 * d
    NBF = V // d2
    TAIL = V - NBF * d2
    info = plsc.get_sparse_core_info()
    NC, NS = info.num_cores, info.num_subcores
    NW = NC * NS
    NI = (NBF + NW - 1) // NW
    mesh = plsc.VectorSubcoreMesh(core_axis_name="c", subcore_axis_name="s")

    @functools.partial(
        pl.kernel,
        mesh=mesh,
        out_type=jax.ShapeDtypeStruct((V // 2, d2), jnp.float32),
        scratch_types=[
            pltpu.VMEM((d, d2), jnp.float32),
            pltpu.VMEM((d, d2), jnp.float32),
            pltpu.VMEM((d, d), jnp.float32),
        ],
        compiler_params=pltpu.CompilerParams(
            use_tc_tiling_on_sc=True, needs_layout_passes=False
        ),
    )
    def transpose_kernel(et_hbm, out_hbm, buf_v, trans_v, tail_v):
        wid = lax.axis_index("s") * NC + lax.axis_index("c")
        lanes = lax.iota(jnp.int32, 16)
        sj = []
        hi = []
        for jg in range(d2 // 16):
            jv = jg * 16 + lanes
            sj.append(jv & (d - 1))
            hi.append((jv >= d).astype(jnp.int32))

        def do_block(b, rows, width, src_buf):
            pltpu.sync_copy(et_hbm.at[:, pl.ds(b * d2, width)], src_buf)
            for m in range(rows):
                for jg in range(d2 // 16):
                    cvec = hi[jg] + (2 * m)
                    vals = plsc.load_gather(src_buf, [sj[jg], cvec])
                    trans_v[m, pl.ds(jg * 16, 16)] = vals
            pltpu.sync_copy(
                trans_v.at[pl.ds(0, rows)], out_hbm.at[pl.ds(b * d, rows)]
            )

        def loop_body(i, carry):
            b = wid + i * NW

            @pl.when(b < NBF)
            def _():
                do_block(b, d, d2, buf_v)

            return carry

        lax.fori_loop(0, NI, loop_body, 0)

        if TAIL:
            @pl.when(wid == NW - 1)
            def _():
                do_block(NBF, TAIL // 2, TAIL, tail_v)

    return transpose_kernel(ET)


def _sc_gather_pairs(idx2, table2):
    (BT,) = idx2.shape
    _, d2 = table2.shape
    info = plsc.get_sparse_core_info()
    NC, NS = info.num_cores, info.num_subcores
    NW = NC * NS
    per_w = BT // NW
    CHUNK = 320
    NCH = per_w // CHUNK
    mesh = plsc.VectorSubcoreMesh(core_axis_name="c", subcore_axis_name="s")

    @functools.partial(
        pl.kernel,
        mesh=mesh,
        out_type=jax.ShapeDtypeStruct((BT, d2), jnp.float32),
        scratch_types=[
            pltpu.VMEM((per_w,), jnp.int32),
            pltpu.VMEM((CHUNK, d2), jnp.float32),
            pltpu.SemaphoreType.DMA,
        ],
        compiler_params=pltpu.CompilerParams(use_tc_tiling_on_sc=True),
    )
    def gather_kernel(idx_hbm, table_hbm, out_hbm, idx_v, rows_v, sem):
        wid = lax.axis_index("s") * NC + lax.axis_index("c")
        base = wid * per_w
        pltpu.sync_copy(idx_hbm.at[pl.ds(base, per_w)], idx_v)
        for ch in range(NCH):
            pltpu.async_copy(
                table_hbm.at[idx_v.at[pl.ds(ch * CHUNK, CHUNK)]], rows_v, sem
            ).wait()
            pltpu.sync_copy(rows_v, out_hbm.at[pl.ds(base + ch * CHUNK, CHUNK)])

    return gather_kernel(idx2, table2)


def _lstm_tc(emb2_tm, xb, W, U, b2):
    T, B, d2 = emb2_tm.shape
    d = d2 // 2
    u = U.shape[0]
    G = 4 * u
    bB = 512
    grid = (B // bB,)

    def body(emb_ref, x_ref, w_ref, u_ref, b_ref, out_ref, h_ref, c_ref):
        Wm = w_ref[...]
        Um = u_ref[...]
        bv = b_ref[...]
        h = jnp.zeros((bB, u), jnp.float32)
        c = jnp.zeros((bB, u), jnp.float32)
        for t in range(T):
            row = emb_ref[t]
            p = x_ref[:, t:t + 1] & 1
            xt = jnp.where(p == 1, row[:, d:], row[:, :d])
            z = (jnp.dot(xt, Wm, preferred_element_type=jnp.float32)
                 + jnp.dot(h, Um, preferred_element_type=jnp.float32) + bv)
            i = jax.nn.sigmoid(z[:, :u])
            f = jax.nn.sigmoid(z[:, u:2 * u])
            g = jnp.tanh(z[:, 2 * u:3 * u])
            o = jax.nn.sigmoid(z[:, 3 * u:])
            c = f * c + i * g
            h = o * jnp.tanh(c)
            out_ref[t] = h
        h_ref[...] = h
        c_ref[...] = c

    return pl.pallas_call(
        body,
        grid=grid,
        in_specs=[
            pl.BlockSpec((T, bB, d2), lambda i: (0, i, 0)),
            pl.BlockSpec((bB, T), lambda i: (i, 0)),
            pl.BlockSpec((d, G), lambda i: (0, 0)),
            pl.BlockSpec((u, G), lambda i: (0, 0)),
            pl.BlockSpec((1, G), lambda i: (0, 0)),
        ],
        out_specs=[
            pl.BlockSpec((T, bB, u), lambda i: (0, i, 0)),
            pl.BlockSpec((bB, u), lambda i: (i, 0)),
            pl.BlockSpec((bB, u), lambda i: (i, 0)),
        ],
        out_shape=[
            jax.ShapeDtypeStruct((T, B, u), jnp.float32),
            jax.ShapeDtypeStruct((B, u), jnp.float32),
            jax.ShapeDtypeStruct((B, u), jnp.float32),
        ],
        compiler_params=pltpu.CompilerParams(
            dimension_semantics=("arbitrary",),
        ),
    )(emb2_tm, xb, W, U, b2)


def kernel(x, E, W, U, b):
    B, T = x.shape
    V, d = E.shape
    table2 = _sc_transpose_pack(jnp.swapaxes(E, 0, 1))
    idx_tm = jnp.swapaxes(x, 0, 1).reshape(-1)
    emb2_flat = _sc_gather_pairs(idx_tm >> 1, table2)
    emb2_tm = emb2_flat.reshape(T, B, 2 * d)
    out_tm, hT, cT = _lstm_tc(emb2_tm, x, W, U, b.reshape(1, -1))
    return jnp.swapaxes(out_tm, 0, 1), hT, cT


# speedup vs baseline: 4.1400x; 4.1400x over previous
import functools

import jax
import jax.numpy as jnp
from jax import lax
from jax.experimental import pallas as pl
from jax.experimental.pallas import tpu as pltpu
from jax.experimental.pallas import tpu_sc as plsc

_C = 2048


def _tc_transpose_pack(ET, ET_tail):
    d, V = ET.shape
    NB = (V + _C - 1) // _C // 2 + 1
    LASTP = (V - _C) // _C

    def body(l_ref, r_ref, tail_ref, out_ref):
        ii = pl.program_id(0)
        out_ref[:, 0:d] = l_ref[...].T

        @pl.when(ii < NB)
        def _():
            out_ref[:, d:2 * d] = r_ref[...].T

        @pl.when(ii == NB)
        def _():
            out_ref[:, d:2 * d] = tail_ref[...].T

    return pl.pallas_call(
        body,
        grid=(NB + 1,),
        in_specs=[
            pl.BlockSpec((d, _C), lambda i: (0, i)),
            pl.BlockSpec(
                (d, _C), lambda i, NBc=NB, LP=LASTP: (0, jnp.minimum(i + NBc, LP))
            ),
            pl.BlockSpec((d, _C), lambda i: (0, 0)),
        ],
        out_specs=pl.BlockSpec((_C, 2 * d), lambda i: (i, 0)),
        out_shape=jax.ShapeDtypeStruct(((NB + 1) * _C, 2 * d), jnp.float32),
        compiler_params=pltpu.CompilerParams(dimension_semantics=("arbitrary",)),
    )(ET, ET, ET_tail)


def _sc_gather_pairs(idx2, table2):
    (BT,) = idx2.shape
    _, d2 = table2.shape
    info = plsc.get_sparse_core_info()
    NC, NS = info.num_cores, info.num_subcores
    NW = NC * NS
    per_w = BT // NW
    CHUNK = 320
    NCH = per_w // CHUNK
    mesh = plsc.VectorSubcoreMesh(core_axis_name="c", subcore_axis_name="s")

    @functools.partial(
        pl.kernel,
        mesh=mesh,
        out_type=jax.ShapeDtypeStruct((BT, d2), jnp.float32),
        scratch_types=[
            pltpu.VMEM((per_w,), jnp.int32),
            pltpu.VMEM((CHUNK, d2), jnp.float32),
            pltpu.SemaphoreType.DMA,
        ],
        compiler_params=pltpu.CompilerParams(use_tc_tiling_on_sc=True),
    )
    def gather_kernel(idx_hbm, table_hbm, out_hbm, idx_v, rows_v, sem):
        wid = lax.axis_index("s") * NC + lax.axis_index("c")
        base = wid * per_w
        pltpu.sync_copy(idx_hbm.at[pl.ds(base, per_w)], idx_v)
        for ch in range(NCH):
            pltpu.async_copy(
                table_hbm.at[idx_v.at[pl.ds(ch * CHUNK, CHUNK)]], rows_v, sem
            ).wait()
            pltpu.sync_copy(rows_v, out_hbm.at[pl.ds(base + ch * CHUNK, CHUNK)])

    return gather_kernel(idx2, table2)


def _lstm_tc(emb2_tm, xb, W, U, b2, hsplit):
    T, B, d2 = emb2_tm.shape
    d = d2 // 2
    u = U.shape[0]
    G = 4 * u
    bB = 512
    grid = (B // bB,)

    def body(emb_ref, x_ref, w_ref, u_ref, b_ref, out_ref, h_ref, c_ref):
        Wm = w_ref[...]
        Um = u_ref[...]
        bv = b_ref[...]
        h = jnp.zeros((bB, u), jnp.float32)
        c = jnp.zeros((bB, u), jnp.float32)
        for t in range(T):
            row = emb_ref[t]
            p = x_ref[:, t:t + 1] >= hsplit
            xt = jnp.where(p, row[:, d:], row[:, :d])
            z = (jnp.dot(xt, Wm, preferred_element_type=jnp.float32)
                 + jnp.dot(h, Um, preferred_element_type=jnp.float32) + bv)
            i = jax.nn.sigmoid(z[:, :u])
            f = jax.nn.sigmoid(z[:, u:2 * u])
            g = jnp.tanh(z[:, 2 * u:3 * u])
            o = jax.nn.sigmoid(z[:, 3 * u:])
            c = f * c + i * g
            h = o * jnp.tanh(c)
            out_ref[t] = h
        h_ref[...] = h
        c_ref[...] = c

    return pl.pallas_call(
        body,
        grid=grid,
        in_specs=[
            pl.BlockSpec((T, bB, d2), lambda i: (0, i, 0)),
            pl.BlockSpec((bB, T), lambda i: (i, 0)),
            pl.BlockSpec((d, G), lambda i: (0, 0)),
            pl.BlockSpec((u, G), lambda i: (0, 0)),
            pl.BlockSpec((1, G), lambda i: (0, 0)),
        ],
        out_specs=[
            pl.BlockSpec((T, bB, u), lambda i: (0, i, 0)),
            pl.BlockSpec((bB, u), lambda i: (i, 0)),
            pl.BlockSpec((bB, u), lambda i: (i, 0)),
        ],
        out_shape=[
            jax.ShapeDtypeStruct((T, B, u), jnp.float32),
            jax.ShapeDtypeStruct((B, u), jnp.float32),
            jax.ShapeDtypeStruct((B, u), jnp.float32),
        ],
        compiler_params=pltpu.CompilerParams(
            dimension_semantics=("arbitrary",),
        ),
    )(emb2_tm, xb, W, U, b2)


def kernel(x, E, W, U, b):
    B, T = x.shape
    V, d = E.shape
    NB = (V + _C - 1) // _C // 2 + 1
    HSPLIT = NB * _C
    VFULL = (V // _C) * _C
    TAILSHIFT = HSPLIT - (V - _C - HSPLIT)

    ET = jnp.swapaxes(E, 0, 1)
    table2 = _tc_transpose_pack(ET, lax.slice(ET, (0, V - _C), (d, V)))
    idx_tm = jnp.swapaxes(x, 0, 1).reshape(-1)
    idx2 = (idx_tm
            - jnp.where(idx_tm >= HSPLIT, HSPLIT, 0)
            + jnp.where(idx_tm >= VFULL, TAILSHIFT, 0))
    emb2_flat = _sc_gather_pairs(idx2, table2)
    emb2_tm = emb2_flat.reshape(T, B, 2 * d)
    out_tm, hT, cT = _lstm_tc(emb2_tm, x, W, U, b.reshape(1, -1), HSPLIT)
    return jnp.swapaxes(out_tm, 0, 1), hT, cT


# speedup vs baseline: 4.8362x; 1.1682x over previous
import functools

import jax
import jax.numpy as jnp
from jax import lax
from jax.experimental import pallas as pl
from jax.experimental.pallas import tpu as pltpu
from jax.experimental.pallas import tpu_sc as plsc

_C = 4096


def _tc_transpose_pack(ET, ET_tail):
    d, V = ET.shape
    NB = (V + _C - 1) // _C // 2 + 1
    LASTP = (V - _C) // _C

    def body(l_ref, r_ref, tail_ref, out_ref):
        ii = pl.program_id(0)
        out_ref[:, 0:d] = l_ref[...].T

        @pl.when(ii < NB)
        def _():
            out_ref[:, d:2 * d] = r_ref[...].T

        @pl.when(ii == NB)
        def _():
            out_ref[:, d:2 * d] = tail_ref[...].T

    return pl.pallas_call(
        body,
        grid=(NB + 1,),
        in_specs=[
            pl.BlockSpec((d, _C), lambda i: (0, i)),
            pl.BlockSpec(
                (d, _C), lambda i, NBc=NB, LP=LASTP: (0, jnp.minimum(i + NBc, LP))
            ),
            pl.BlockSpec((d, _C), lambda i: (0, 0)),
        ],
        out_specs=pl.BlockSpec((_C, 2 * d), lambda i: (i, 0)),
        out_shape=jax.ShapeDtypeStruct(((NB + 1) * _C, 2 * d), jnp.float32),
        compiler_params=pltpu.CompilerParams(dimension_semantics=("arbitrary",)),
    )(ET, ET, ET_tail)


def _sc_gather_pairs(idx2, table2):
    (BT,) = idx2.shape
    _, d2 = table2.shape
    info = plsc.get_sparse_core_info()
    NC, NS = info.num_cores, info.num_subcores
    NW = NC * NS
    per_w = BT // NW
    CHUNK = 320
    NCH = per_w // CHUNK
    mesh = plsc.VectorSubcoreMesh(core_axis_name="c", subcore_axis_name="s")

    @functools.partial(
        pl.kernel,
        mesh=mesh,
        out_type=jax.ShapeDtypeStruct((BT, d2), jnp.float32),
        scratch_types=[
            pltpu.VMEM((per_w,), jnp.int32),
            pltpu.VMEM((CHUNK, d2), jnp.float32),
            pltpu.SemaphoreType.DMA,
        ],
        compiler_params=pltpu.CompilerParams(use_tc_tiling_on_sc=True),
    )
    def gather_kernel(idx_hbm, table_hbm, out_hbm, idx_v, rows_v, sem):
        wid = lax.axis_index("s") * NC + lax.axis_index("c")
        base = wid * per_w
        pltpu.sync_copy(idx_hbm.at[pl.ds(base, per_w)], idx_v)
        for ch in range(NCH):
            pltpu.async_copy(
                table_hbm.at[idx_v.at[pl.ds(ch * CHUNK, CHUNK)]], rows_v, sem
            ).wait()
            pltpu.sync_copy(rows_v, out_hbm.at[pl.ds(base + ch * CHUNK, CHUNK)])

    return gather_kernel(idx2, table2)


def _lstm_tc(emb2_tm, xb, W, U, b2, hsplit):
    T, B, d2 = emb2_tm.shape
    d = d2 // 2
    u = U.shape[0]
    G = 4 * u
    bB = 512
    grid = (B // bB,)

    def body(emb_ref, x_ref, w_ref, u_ref, b_ref, out_ref, h_ref, c_ref):
        Wm = w_ref[...]
        Um = u_ref[...]
        bv = b_ref[...]
        h = jnp.zeros((bB, u), jnp.float32)
        c = jnp.zeros((bB, u), jnp.float32)
        for t in range(T):
            row = emb_ref[t]
            p = x_ref[:, t:t + 1] >= hsplit
            xt = jnp.where(p, row[:, d:], row[:, :d])
            z = (jnp.dot(xt, Wm, preferred_element_type=jnp.float32)
                 + jnp.dot(h, Um, preferred_element_type=jnp.float32) + bv)
            i = jax.nn.sigmoid(z[:, :u])
            f = jax.nn.sigmoid(z[:, u:2 * u])
            g = jnp.tanh(z[:, 2 * u:3 * u])
            o = jax.nn.sigmoid(z[:, 3 * u:])
            c = f * c + i * g
            h = o * jnp.tanh(c)
            out_ref[t] = h
        h_ref[...] = h
        c_ref[...] = c

    return pl.pallas_call(
        body,
        grid=grid,
        in_specs=[
            pl.BlockSpec((T, bB, d2), lambda i: (0, i, 0)),
            pl.BlockSpec((bB, T), lambda i: (i, 0)),
            pl.BlockSpec((d, G), lambda i: (0, 0)),
            pl.BlockSpec((u, G), lambda i: (0, 0)),
            pl.BlockSpec((1, G), lambda i: (0, 0)),
        ],
        out_specs=[
            pl.BlockSpec((T, bB, u), lambda i: (0, i, 0)),
            pl.BlockSpec((bB, u), lambda i: (i, 0)),
            pl.BlockSpec((bB, u), lambda i: (i, 0)),
        ],
        out_shape=[
            jax.ShapeDtypeStruct((T, B, u), jnp.float32),
            jax.ShapeDtypeStruct((B, u), jnp.float32),
            jax.ShapeDtypeStruct((B, u), jnp.float32),
        ],
        compiler_params=pltpu.CompilerParams(
            dimension_semantics=("arbitrary",),
        ),
    )(emb2_tm, xb, W, U, b2)


def kernel(x, E, W, U, b):
    B, T = x.shape
    V, d = E.shape
    NB = (V + _C - 1) // _C // 2 + 1
    HSPLIT = NB * _C
    VFULL = (V // _C) * _C
    TAILSHIFT = HSPLIT - (V - _C - HSPLIT)

    ET = jnp.swapaxes(E, 0, 1)
    table2 = _tc_transpose_pack(ET, lax.slice(ET, (0, V - _C), (d, V)))
    idx_tm = jnp.swapaxes(x, 0, 1).reshape(-1)
    idx2 = (idx_tm
            - jnp.where(idx_tm >= HSPLIT, HSPLIT, 0)
            + jnp.where(idx_tm >= VFULL, TAILSHIFT, 0))
    emb2_flat = _sc_gather_pairs(idx2, table2)
    emb2_tm = emb2_flat.reshape(T, B, 2 * d)
    out_tm, hT, cT = _lstm_tc(emb2_tm, x, W, U, b.reshape(1, -1), HSPLIT)
    return jnp.swapaxes(out_tm, 0, 1), hT, cT


# speedup vs baseline: 5.2387x; 1.0832x over previous
import functools

import jax
import jax.numpy as jnp
from jax import lax
from jax.experimental import pallas as pl
from jax.experimental.pallas import tpu as pltpu
from jax.experimental.pallas import tpu_sc as plsc

_C = 8192


def _tc_transpose_pack(ET, ET_tail):
    d, V = ET.shape
    NB = (V + _C - 1) // _C // 2 + 1
    LASTP = (V - _C) // _C

    def body(l_ref, r_ref, tail_ref, out_ref):
        ii = pl.program_id(0)
        out_ref[:, 0:d] = l_ref[...].T

        @pl.when(ii < NB)
        def _():
            out_ref[:, d:2 * d] = r_ref[...].T

        @pl.when(ii == NB)
        def _():
            out_ref[:, d:2 * d] = tail_ref[...].T

    return pl.pallas_call(
        body,
        grid=(NB + 1,),
        in_specs=[
            pl.BlockSpec((d, _C), lambda i: (0, i)),
            pl.BlockSpec(
                (d, _C), lambda i, NBc=NB, LP=LASTP: (0, jnp.minimum(i + NBc, LP))
            ),
            pl.BlockSpec((d, _C), lambda i: (0, 0)),
        ],
        out_specs=pl.BlockSpec((_C, 2 * d), lambda i: (i, 0)),
        out_shape=jax.ShapeDtypeStruct(((NB + 1) * _C, 2 * d), jnp.float32),
        compiler_params=pltpu.CompilerParams(dimension_semantics=("arbitrary",)),
    )(ET, ET, ET_tail)


def _sc_gather_pairs(idx2, table2):
    (BT,) = idx2.shape
    _, d2 = table2.shape
    info = plsc.get_sparse_core_info()
    NC, NS = info.num_cores, info.num_subcores
    NW = NC * NS
    per_w = BT // NW
    CHUNK = 320
    NCH = per_w // CHUNK
    mesh = plsc.VectorSubcoreMesh(core_axis_name="c", subcore_axis_name="s")

    @functools.partial(
        pl.kernel,
        mesh=mesh,
        out_type=jax.ShapeDtypeStruct((BT, d2), jnp.float32),
        scratch_types=[
            pltpu.VMEM((per_w,), jnp.int32),
            pltpu.VMEM((CHUNK, d2), jnp.float32),
            pltpu.SemaphoreType.DMA,
        ],
        compiler_params=pltpu.CompilerParams(use_tc_tiling_on_sc=True),
    )
    def gather_kernel(idx_hbm, table_hbm, out_hbm, idx_v, rows_v, sem):
        wid = lax.axis_index("s") * NC + lax.axis_index("c")
        base = wid * per_w
        pltpu.sync_copy(idx_hbm.at[pl.ds(base, per_w)], idx_v)
        for ch in range(NCH):
            pltpu.async_copy(
                table_hbm.at[idx_v.at[pl.ds(ch * CHUNK, CHUNK)]], rows_v, sem
            ).wait()
            pltpu.sync_copy(rows_v, out_hbm.at[pl.ds(base + ch * CHUNK, CHUNK)])

    return gather_kernel(idx2, table2)


def _lstm_tc(emb2_tm, xb, W, U, b2, hsplit):
    T, B, d2 = emb2_tm.shape
    d = d2 // 2
    u = U.shape[0]
    G = 4 * u
    bB = 512
    grid = (B // bB,)

    def body(emb_ref, x_ref, w_ref, u_ref, b_ref, out_ref, h_ref, c_ref):
        Wm = w_ref[...]
        Um = u_ref[...]
        bv = b_ref[...]
        h = jnp.zeros((bB, u), jnp.float32)
        c = jnp.zeros((bB, u), jnp.float32)
        for t in range(T):
            row = emb_ref[t]
            p = x_ref[:, t:t + 1] >= hsplit
            xt = jnp.where(p, row[:, d:], row[:, :d])
            z = (jnp.dot(xt, Wm, preferred_element_type=jnp.float32)
                 + jnp.dot(h, Um, preferred_element_type=jnp.float32) + bv)
            i = jax.nn.sigmoid(z[:, :u])
            f = jax.nn.sigmoid(z[:, u:2 * u])
            g = jnp.tanh(z[:, 2 * u:3 * u])
            o = jax.nn.sigmoid(z[:, 3 * u:])
            c = f * c + i * g
            h = o * jnp.tanh(c)
            out_ref[t] = h
        h_ref[...] = h
        c_ref[...] = c

    return pl.pallas_call(
        body,
        grid=grid,
        in_specs=[
            pl.BlockSpec((T, bB, d2), lambda i: (0, i, 0)),
            pl.BlockSpec((bB, T), lambda i: (i, 0)),
            pl.BlockSpec((d, G), lambda i: (0, 0)),
            pl.BlockSpec((u, G), lambda i: (0, 0)),
            pl.BlockSpec((1, G), lambda i: (0, 0)),
        ],
        out_specs=[
            pl.BlockSpec((T, bB, u), lambda i: (0, i, 0)),
            pl.BlockSpec((bB, u), lambda i: (i, 0)),
            pl.BlockSpec((bB, u), lambda i: (i, 0)),
        ],
        out_shape=[
            jax.ShapeDtypeStruct((T, B, u), jnp.float32),
            jax.ShapeDtypeStruct((B, u), jnp.float32),
            jax.ShapeDtypeStruct((B, u), jnp.float32),
        ],
        compiler_params=pltpu.CompilerParams(
            dimension_semantics=("arbitrary",),
        ),
    )(emb2_tm, xb, W, U, b2)


def kernel(x, E, W, U, b):
    B, T = x.shape
    V, d = E.shape
    NB = (V + _C - 1) // _C // 2 + 1
    HSPLIT = NB * _C
    VFULL = (V // _C) * _C
    TAILSHIFT = HSPLIT - (V - _C - HSPLIT)

    ET = jnp.swapaxes(E, 0, 1)
    table2 = _tc_transpose_pack(ET, lax.slice(ET, (0, V - _C), (d, V)))
    idx_tm = jnp.swapaxes(x, 0, 1).reshape(-1)
    idx2 = (idx_tm
            - jnp.where(idx_tm >= HSPLIT, HSPLIT, 0)
            + jnp.where(idx_tm >= VFULL, TAILSHIFT, 0))
    emb2_flat = _sc_gather_pairs(idx2, table2)
    emb2_tm = emb2_flat.reshape(T, B, 2 * d)
    out_tm, hT, cT = _lstm_tc(emb2_tm, x, W, U, b.reshape(1, -1), HSPLIT)
    return jnp.swapaxes(out_tm, 0, 1), hT, cT


# speedup vs baseline: 5.2704x; 1.0060x over previous
import functools

import jax
import jax.numpy as jnp
from jax import lax
from jax.experimental import pallas as pl
from jax.experimental.pallas import tpu as pltpu
from jax.experimental.pallas import tpu_sc as plsc

_C = 16384


def _tc_transpose_pack(ET, ET_tail):
    d, V = ET.shape
    NB = (V + _C - 1) // _C // 2 + 1
    LASTP = (V - _C) // _C

    def body(l_ref, r_ref, tail_ref, out_ref):
        ii = pl.program_id(0)
        out_ref[:, 0:d] = l_ref[...].T

        @pl.when(ii < NB)
        def _():
            out_ref[:, d:2 * d] = r_ref[...].T

        @pl.when(ii == NB)
        def _():
            out_ref[:, d:2 * d] = tail_ref[...].T

    return pl.pallas_call(
        body,
        grid=(NB + 1,),
        in_specs=[
            pl.BlockSpec((d, _C), lambda i: (0, i)),
            pl.BlockSpec(
                (d, _C), lambda i, NBc=NB, LP=LASTP: (0, jnp.minimum(i + NBc, LP))
            ),
            pl.BlockSpec((d, _C), lambda i: (0, 0)),
        ],
        out_specs=pl.BlockSpec((_C, 2 * d), lambda i: (i, 0)),
        out_shape=jax.ShapeDtypeStruct(((NB + 1) * _C, 2 * d), jnp.float32),
        compiler_params=pltpu.CompilerParams(dimension_semantics=("arbitrary",)),
    )(ET, ET, ET_tail)


def _sc_gather_pairs(idx2, table2):
    (BT,) = idx2.shape
    _, d2 = table2.shape
    info = plsc.get_sparse_core_info()
    NC, NS = info.num_cores, info.num_subcores
    NW = NC * NS
    per_w = BT // NW
    CHUNK = 320
    NCH = per_w // CHUNK
    mesh = plsc.VectorSubcoreMesh(core_axis_name="c", subcore_axis_name="s")

    @functools.partial(
        pl.kernel,
        mesh=mesh,
        out_type=jax.ShapeDtypeStruct((BT, d2), jnp.float32),
        scratch_types=[
            pltpu.VMEM((per_w,), jnp.int32),
            pltpu.VMEM((CHUNK, d2), jnp.float32),
            pltpu.SemaphoreType.DMA,
        ],
        compiler_params=pltpu.CompilerParams(use_tc_tiling_on_sc=True),
    )
    def gather_kernel(idx_hbm, table_hbm, out_hbm, idx_v, rows_v, sem):
        wid = lax.axis_index("s") * NC + lax.axis_index("c")
        base = wid * per_w
        pltpu.sync_copy(idx_hbm.at[pl.ds(base, per_w)], idx_v)
        for ch in range(NCH):
            pltpu.async_copy(
                table_hbm.at[idx_v.at[pl.ds(ch * CHUNK, CHUNK)]], rows_v, sem
            ).wait()
            pltpu.sync_copy(rows_v, out_hbm.at[pl.ds(base + ch * CHUNK, CHUNK)])

    return gather_kernel(idx2, table2)


def _lstm_tc(emb2_tm, xb, W, U, b2, hsplit):
    T, B, d2 = emb2_tm.shape
    d = d2 // 2
    u = U.shape[0]
    G = 4 * u
    bB = 512
    grid = (B // bB,)

    def body(emb_ref, x_ref, w_ref, u_ref, b_ref, out_ref, h_ref, c_ref):
        Wm = w_ref[...]
        Um = u_ref[...]
        bv = b_ref[...]
        h = jnp.zeros((bB, u), jnp.float32)
        c = jnp.zeros((bB, u), jnp.float32)
        for t in range(T):
            row = emb_ref[t]
            p = x_ref[:, t:t + 1] >= hsplit
            xt = jnp.where(p, row[:, d:], row[:, :d])
            z = (jnp.dot(xt, Wm, preferred_element_type=jnp.float32)
                 + jnp.dot(h, Um, preferred_element_type=jnp.float32) + bv)
            i = jax.nn.sigmoid(z[:, :u])
            f = jax.nn.sigmoid(z[:, u:2 * u])
            g = jnp.tanh(z[:, 2 * u:3 * u])
            o = jax.nn.sigmoid(z[:, 3 * u:])
            c = f * c + i * g
            h = o * jnp.tanh(c)
            out_ref[t] = h
        h_ref[...] = h
        c_ref[...] = c

    return pl.pallas_call(
        body,
        grid=grid,
        in_specs=[
            pl.BlockSpec((T, bB, d2), lambda i: (0, i, 0)),
            pl.BlockSpec((bB, T), lambda i: (i, 0)),
            pl.BlockSpec((d, G), lambda i: (0, 0)),
            pl.BlockSpec((u, G), lambda i: (0, 0)),
            pl.BlockSpec((1, G), lambda i: (0, 0)),
        ],
        out_specs=[
            pl.BlockSpec((T, bB, u), lambda i: (0, i, 0)),
            pl.BlockSpec((bB, u), lambda i: (i, 0)),
            pl.BlockSpec((bB, u), lambda i: (i, 0)),
        ],
        out_shape=[
            jax.ShapeDtypeStruct((T, B, u), jnp.float32),
            jax.ShapeDtypeStruct((B, u), jnp.float32),
            jax.ShapeDtypeStruct((B, u), jnp.float32),
        ],
        compiler_params=pltpu.CompilerParams(
            dimension_semantics=("arbitrary",),
        ),
    )(emb2_tm, xb, W, U, b2)


def kernel(x, E, W, U, b):
    B, T = x.shape
    V, d = E.shape
    NB = (V + _C - 1) // _C // 2 + 1
    HSPLIT = NB * _C
    VFULL = (V // _C) * _C
    TAILSHIFT = HSPLIT - (V - _C - HSPLIT)

    ET = jnp.swapaxes(E, 0, 1)
    table2 = _tc_transpose_pack(ET, lax.slice(ET, (0, V - _C), (d, V)))
    idx_tm = jnp.swapaxes(x, 0, 1).reshape(-1)
    idx2 = (idx_tm
            - jnp.where(idx_tm >= HSPLIT, HSPLIT, 0)
            + jnp.where(idx_tm >= VFULL, TAILSHIFT, 0))
    emb2_flat = _sc_gather_pairs(idx2, table2)
    emb2_tm = emb2_flat.reshape(T, B, 2 * d)
    out_tm, hT, cT = _lstm_tc(emb2_tm, x, W, U, b.reshape(1, -1), HSPLIT)
    return jnp.swapaxes(out_tm, 0, 1), hT, cT
